# baseline (speedup 1.0000x reference)
import functools

import jax
import jax.numpy as jnp
from jax import lax
from jax.experimental import pallas as pl
from jax.experimental.pallas import tpu as pltpu
from jax.experimental.pallas import tpu_sc as plsc

_B = 4
_S = 4096
_D = 1024
_NC = 2
_NS = 16
_NW = _NC * _NS
_PPW = _S // _NW
_C = 8
_NCHUNK = _PPW // _C
_LANES = 16


def _body(tokens_hbm, table_hbm, pos_hbm, out_hbm, idx_v, pos_v, rows_v, gsem):
    wid = lax.axis_index("s") * _NC + lax.axis_index("c")
    p0 = wid * _PPW

    for b in range(_B):
        pltpu.sync_copy(tokens_hbm.at[b, pl.ds(p0, _PPW)], idx_v.at[b])

    def chunk(ci, carry):
        s0 = p0 + ci * _C
        c0 = ci * _C
        pltpu.sync_copy(pos_hbm.at[pl.ds(s0, _C)], pos_v)
        for b in range(_B):
            pltpu.async_copy(
                table_hbm.at[idx_v.at[b, pl.ds(c0, _C)]],
                rows_v.at[pl.ds(b * _C, _C)],
                gsem,
            ).wait()

        def addloop(i, c2):
            off = i * _LANES
            for c in range(_C):
                pv = pos_v[c, pl.ds(off, _LANES)]
                for b in range(_B):
                    r = b * _C + c
                    rows_v[r, pl.ds(off, _LANES)] = (
                        rows_v[r, pl.ds(off, _LANES)] + pv
                    )
            return c2

        lax.fori_loop(0, _D // _LANES, addloop, 0)

        for b in range(_B):
            pltpu.sync_copy(
                rows_v.at[pl.ds(b * _C, _C)], out_hbm.at[b, pl.ds(s0, _C)]
            )
        return carry

    lax.fori_loop(0, _NCHUNK, chunk, 0)


@jax.jit
def _emb(tokens, table, pos2d):
    mesh = plsc.VectorSubcoreMesh(core_axis_name="c", subcore_axis_name="s")
    return pl.kernel(
        _body,
        out_type=jax.ShapeDtypeStruct((_B, _S, _D), jnp.float32),
        mesh=mesh,
        scratch_types=[
            pltpu.VMEM((_B, _PPW), jnp.int32),
            pltpu.VMEM((_C, _D), jnp.float32),
            pltpu.VMEM((_B * _C, _D), jnp.float32),
            pltpu.SemaphoreType.DMA,
        ],
    )(tokens, table, pos2d)


def kernel(tokens, table, pos):
    tokens = tokens.astype(jnp.int32)
    pos2d = pos.reshape(pos.shape[1], pos.shape[2])[: tokens.shape[1]]
    return _emb(tokens, table, pos2d)
